# baseline (speedup 1.0000x reference)
import functools
import jax
import jax.numpy as jnp
from jax import lax
from jax.experimental import pallas as pl
from jax.experimental.pallas import tpu as pltpu
from jax.experimental.pallas import tpu_sc as plsc

N = 10000
E = 320000
D = 128

NC = 2
NS = 16
NW = NC * NS
EPT = E // NW
CHUNK = 128
NP = 10240
HR = NP // 128
BLK = 2040
NPASS = 5
ACC_R = 2047
CAP = 10752
CAPC = CAP // CHUNK

_mesh = plsc.VectorSubcoreMesh(core_axis_name="c", subcore_axis_name="s")
_sc_params = pltpu.CompilerParams(needs_layout_passes=False)


@functools.partial(
    pl.kernel,
    mesh=_mesh,
    compiler_params=_sc_params,
    out_type=(
        jax.ShapeDtypeStruct((NW, HR, 128), jnp.float32),
        jax.ShapeDtypeStruct((NW, HR, 128), jnp.float32),
        jax.ShapeDtypeStruct((NW, CAPC, CHUNK), jnp.int32),
        jax.ShapeDtypeStruct((NW, CAPC, CHUNK), jnp.int32),
        jax.ShapeDtypeStruct((NW, 1, 16), jnp.int32),
    ),
    scratch_types=[
        pltpu.VMEM((EPT,), jnp.int32),
        pltpu.VMEM((EPT,), jnp.int32),
        pltpu.VMEM((HR, 128), jnp.float32),
        pltpu.VMEM((CAPC, CHUNK), jnp.int32),
        pltpu.VMEM((CAPC, CHUNK), jnp.int32),
        pltpu.VMEM((1, 16), jnp.int32),
    ],
)
def _prep_kernel(src_hbm, dst_hbm, dego_hbm, degi_hbm, psrc_hbm, pdst_hbm,
                 cnt_hbm, src_v, dst_v, hist, stg_s, stg_d, cnt_v):
    c = lax.axis_index("c")
    s = lax.axis_index("s")
    wid = c * NS + s

    pltpu.sync_copy(src_hbm.at[pl.ds(wid * EPT, EPT)], src_v)
    pltpu.sync_copy(dst_hbm.at[pl.ds(wid * EPT, EPT)], dst_v)

    def zero_hist():
        def zf(k, _):
            hist[k // 8, pl.ds((k % 8) * 16, 16)] = jnp.zeros(
                (16,), jnp.float32)
            return 0

        lax.fori_loop(0, HR * 8, zf, 0)

    def count_hist(edges_v, out_hbm):
        zero_hist()

        def body(k, _):
            for u in range(4):
                v = edges_v[pl.ds(k * 64 + u * 16, 16)]
                cnts, last = plsc.scan_count(v)
                plsc.addupdate_scatter(
                    hist,
                    [lax.shift_right_logical(v, 7), v & 127],
                    cnts.astype(jnp.float32),
                    mask=last,
                )
            return 0

        lax.fori_loop(0, EPT // 64, body, 0)
        for k in range(EPT // 64 * 4, EPT // 16):
            v = edges_v[pl.ds(k * 16, 16)]
            cnts, last = plsc.scan_count(v)
            plsc.addupdate_scatter(
                hist,
                [lax.shift_right_logical(v, 7), v & 127],
                cnts.astype(jnp.float32),
                mask=last,
            )
        pltpu.sync_copy(hist, out_hbm.at[wid])

    count_hist(src_v, dego_hbm)
    count_hist(dst_v, degi_hbm)

    trash = BLK + (s % (ACC_R - BLK))
    lanes = lax.iota(jnp.int32, 16)
    cnt = jnp.int32(0)
    prev_chunks = jnp.int32(0)
    counts_vec = jnp.zeros((16,), jnp.int32)
    for p in range(NPASS):
        lo = jnp.int32(p * BLK)

        def scan_one(k, cnt):
            d = dst_v[pl.ds(k * 16, 16)]
            sv = src_v[pl.ds(k * 16, 16)]
            dl = d - lo
            m = (dl >= 0) & (dl < BLK)
            mi = m.astype(jnp.int32)
            pos = cnt + plsc.cumsum(mi) - mi
            pr = pos // CHUNK
            pc = pos - pr * CHUNK
            plsc.store_scatter(stg_s, [pr, pc], sv, mask=m)
            plsc.store_scatter(stg_d, [pr, pc], dl, mask=m)
            return cnt + jnp.sum(mi)

        def scan(k, cnt):
            for u in range(4):
                cnt = scan_one(k * 4 + u, cnt)
            return cnt

        cnt = lax.fori_loop(0, EPT // 64, scan, cnt)
        for k in range(EPT // 64 * 4, EPT // 16):
            cnt = scan_one(k, cnt)
        for t in range(CHUNK // 16):
            pos = cnt + t * 16 + lanes
            pr = pos // CHUNK
            pc = pos - pr * CHUNK
            plsc.store_scatter(stg_s, [pr, pc], s * CHUNK + t * 16 + lanes)
            plsc.store_scatter(stg_d, [pr, pc],
                               jnp.full((16,), trash, jnp.int32))
        nch = (cnt + (CHUNK - 1)) // CHUNK
        counts_vec = jnp.where(lanes == p, nch - prev_chunks, counts_vec)
        prev_chunks = nch
        cnt = nch * CHUNK
    cnt_v[0, :] = counts_vec

    pltpu.sync_copy(stg_s, psrc_hbm.at[wid])
    pltpu.sync_copy(stg_d, pdst_hbm.at[wid])
    pltpu.sync_copy(cnt_v, cnt_hbm.at[wid])


@functools.partial(
    pl.kernel,
    mesh=_mesh,
    compiler_params=_sc_params,
    out_type=jax.ShapeDtypeStruct((NC * NPASS * BLK, D), jnp.float32),
    scratch_types=[
        pltpu.VMEM((CAPC, CHUNK), jnp.int32),
        pltpu.VMEM((CAPC, CHUNK), jnp.int32),
        pltpu.VMEM((1, 16), jnp.int32),
        pltpu.VMEM((CHUNK, D), jnp.float32),
        pltpu.VMEM((CHUNK, D), jnp.float32),
        pltpu.VMEM((8, D), jnp.float32),
        pltpu.VMEM_SHARED((ACC_R, D), jnp.float32),
        pltpu.SemaphoreType.DMA,
        pltpu.SemaphoreType.DMA,
    ],
)
def _agg_kernel(h_hbm, psrc_hbm, pdst_hbm, cnt_hbm, out_hbm,
                src_v, dst_v, cnt_v, rows_a, rows_b, zbuf, acc, sem_a, sem_b):
    c = lax.axis_index("c")
    s = lax.axis_index("s")
    wid = c * NS + s

    def zfill(k, _):
        zbuf[k // 8, pl.ds((k % 8) * 16, 16)] = jnp.zeros((16,), jnp.float32)
        return 0

    lax.fori_loop(0, 8 * D // 16, zfill, 0)

    pltpu.sync_copy(psrc_hbm.at[wid], src_v)
    pltpu.sync_copy(pdst_hbm.at[wid], dst_v)
    pltpu.sync_copy(cnt_hbm.at[wid], cnt_v)

    slab0 = s * 128
    nslab8 = jnp.where(s < 15, 16, 15)

    def zslab(i, _):
        pltpu.sync_copy(zbuf, acc.at[pl.ds(slab0 + i * 8, 8)])
        return 0

    def zero_slab():
        lax.fori_loop(0, nslab8, zslab, 0)

    zero_slab()
    plsc.subcore_barrier()

    off = jnp.int32(0)
    cv = cnt_v[0, :]

    def gather_start(j, buf, sem):
        return pltpu.async_copy(h_hbm.at[src_v.at[j]], buf, sem)

    for p in range(NPASS):
        nch = cv[p]

        @pl.when(nch > 0)
        def _():
            gather_start(off, rows_a, sem_a)

        def body(jj, off):
            j = off + jj

            def step(buf, sem, nbuf, nsem):
                pltpu.make_async_copy(h_hbm.at[src_v.at[j]], buf, sem).wait()

                @pl.when(jj + 1 < nch)
                def _():
                    gather_start(j + 1, nbuf, nsem)

                pltpu.sync_copy(buf, acc.at[dst_v.at[j]], add=True)

            @pl.when(jj % 2 == 0)
            def _():
                step(rows_a, sem_a, rows_b, sem_b)

            @pl.when(jj % 2 == 1)
            def _():
                step(rows_b, sem_b, rows_a, sem_a)

            return off

        lax.fori_loop(0, nch, body, off)
        off = off + nch
        plsc.subcore_barrier()

        @pl.when(s < 15)
        def _():
            pltpu.sync_copy(
                acc.at[pl.ds(slab0, 128)],
                out_hbm.at[pl.ds(c * NPASS * BLK + p * BLK + slab0, 128)])

        @pl.when(s == 15)
        def _():
            pltpu.sync_copy(
                acc.at[pl.ds(slab0, 120)],
                out_hbm.at[pl.ds(c * NPASS * BLK + p * BLK + slab0, 120)])

        plsc.subcore_barrier()
        if p != NPASS - 1:
            zero_slab()
            plsc.subcore_barrier()


def _elu(t):
    return jnp.where(t > 0, t, jnp.exp(jnp.minimum(t, 0.0)) - 1.0)


def _deg_reduce_body(po_ref, pi_ref, do_ref, di_ref):
    acc_o = jnp.zeros((HR, 128), jnp.float32)
    acc_i = jnp.zeros((HR, 128), jnp.float32)
    for w in range(NW):
        acc_o = acc_o + po_ref[pl.ds(w * HR, HR), :]
        acc_i = acc_i + pi_ref[pl.ds(w * HR, HR), :]
    do_ref[...] = lax.rsqrt(jnp.maximum(acc_o, 1.0))
    di_ref[...] = lax.rsqrt(jnp.maximum(acc_i, 1.0))


_deg_reduce = pl.pallas_call(
    _deg_reduce_body,
    grid=(1,),
    in_specs=[pl.BlockSpec((NW * HR, 128), lambda i: (0, 0)),
              pl.BlockSpec((NW * HR, 128), lambda i: (0, 0))],
    out_specs=[pl.BlockSpec((HR, 128), lambda i: (0, 0)),
               pl.BlockSpec((HR, 128), lambda i: (0, 0))],
    out_shape=[jax.ShapeDtypeStruct((HR, 128), jnp.float32),
               jax.ShapeDtypeStruct((HR, 128), jnp.float32)],
)


def _pre_mm_body(x_ref, do_ref, w_ref, o_ref):
    o_ref[...] = jnp.dot(x_ref[...] * do_ref[...], w_ref[...],
                         preferred_element_type=jnp.float32)


def _mid_mm_body(a0, a1, di_ref, do_ref, b_ref, w_ref, o_ref):
    agg = a0[...] + a1[...]
    t = agg * di_ref[...] + b_ref[...]
    h = _elu(t)
    o_ref[...] = jnp.dot(h * do_ref[...], w_ref[...],
                         preferred_element_type=jnp.float32)


def _final_body(a0, a1, di_ref, b_ref, o_ref):
    agg = a0[...] + a1[...]
    o_ref[...] = _elu(agg * di_ref[...] + b_ref[...])


_BLKR = 1000
_row_spec = pl.BlockSpec((_BLKR, D), lambda i: (i, 0))
_col_spec = pl.BlockSpec((_BLKR, 1), lambda i: (i, 0))
_w_spec = pl.BlockSpec((D, D), lambda i: (0, 0))
_b_spec = pl.BlockSpec((1, D), lambda i: (0, 0))
_out_sds = jax.ShapeDtypeStruct((N, D), jnp.float32)

_pre_mm = pl.pallas_call(
    _pre_mm_body,
    grid=(N // _BLKR,),
    in_specs=[_row_spec, _col_spec, _w_spec],
    out_specs=_row_spec,
    out_shape=_out_sds,
)

_mid_mm = pl.pallas_call(
    _mid_mm_body,
    grid=(N // _BLKR,),
    in_specs=[_row_spec, _row_spec, _col_spec, _col_spec, _b_spec, _w_spec],
    out_specs=_row_spec,
    out_shape=_out_sds,
)

_final = pl.pallas_call(
    _final_body,
    grid=(N // _BLKR,),
    in_specs=[_row_spec, _row_spec, _col_spec, _b_spec],
    out_specs=_row_spec,
    out_shape=_out_sds,
)


def kernel(features, edge_index, W1, b1, W2, b2):
    dego, degi, psrc, pdst, cnts = _prep_kernel(edge_index[0], edge_index[1])

    do_t, di_t = _deg_reduce(dego.reshape(NW * HR, 128),
                             degi.reshape(NW * HR, 128))
    do = do_t.reshape(NP, 1)[:N]
    di = di_t.reshape(NP, 1)[:N]

    h1 = _pre_mm(features, do, W1)
    a1 = _agg_kernel(h1, psrc, pdst, cnts)
    spc = NPASS * BLK
    h2 = _mid_mm(a1[:N], a1[spc:spc + N], di, do, b1.reshape(1, D), W2)
    a2 = _agg_kernel(h2, psrc, pdst, cnts)
    out = _final(a2[:N], a2[spc:spc + N], di, b2.reshape(1, D))
    return out


# speedup vs baseline: 1.1671x; 1.1671x over previous
import functools
import jax
import jax.numpy as jnp
from jax import lax
from jax.experimental import pallas as pl
from jax.experimental.pallas import tpu as pltpu
from jax.experimental.pallas import tpu_sc as plsc

N = 10000
E = 320000
D = 128

NC = 2
NS = 16
NW = NC * NS
EPT = E // NW
CHUNK = 128
NP = 10240
HR = NP // 128
BLK = 2040
NPASS = 5
ACC_R = 2047
CAP = 10752
CAPC = CAP // CHUNK

_mesh = plsc.VectorSubcoreMesh(core_axis_name="c", subcore_axis_name="s")
_sc_params = pltpu.CompilerParams(needs_layout_passes=False)


@functools.partial(
    pl.kernel,
    mesh=_mesh,
    compiler_params=_sc_params,
    out_type=(
        jax.ShapeDtypeStruct((NW, HR, 128), jnp.float32),
        jax.ShapeDtypeStruct((NW, HR, 128), jnp.float32),
        jax.ShapeDtypeStruct((NW, CAPC, CHUNK), jnp.int32),
        jax.ShapeDtypeStruct((NW, CAPC, CHUNK), jnp.int32),
        jax.ShapeDtypeStruct((NW, 1, 16), jnp.int32),
    ),
    scratch_types=[
        pltpu.VMEM((EPT,), jnp.int32),
        pltpu.VMEM((EPT,), jnp.int32),
        pltpu.VMEM((HR, 128), jnp.float32),
        pltpu.VMEM((CAPC, CHUNK), jnp.int32),
        pltpu.VMEM((CAPC, CHUNK), jnp.int32),
        pltpu.VMEM((1, 16), jnp.int32),
    ],
)
def _prep_kernel(src_hbm, dst_hbm, dego_hbm, degi_hbm, psrc_hbm, pdst_hbm,
                 cnt_hbm, src_v, dst_v, hist, stg_s, stg_d, cnt_v):
    c = lax.axis_index("c")
    s = lax.axis_index("s")
    wid = c * NS + s

    pltpu.sync_copy(src_hbm.at[pl.ds(wid * EPT, EPT)], src_v)
    pltpu.sync_copy(dst_hbm.at[pl.ds(wid * EPT, EPT)], dst_v)

    def zero_hist():
        def zf(k, _):
            hist[k // 8, pl.ds((k % 8) * 16, 16)] = jnp.zeros(
                (16,), jnp.float32)
            return 0

        lax.fori_loop(0, HR * 8, zf, 0)

    def count_hist(edges_v, out_hbm):
        zero_hist()

        def body(k, _):
            for u in range(4):
                v = edges_v[pl.ds(k * 64 + u * 16, 16)]
                cnts, last = plsc.scan_count(v)
                plsc.addupdate_scatter(
                    hist,
                    [lax.shift_right_logical(v, 7), v & 127],
                    cnts.astype(jnp.float32),
                    mask=last,
                )
            return 0

        lax.fori_loop(0, EPT // 64, body, 0)
        for k in range(EPT // 64 * 4, EPT // 16):
            v = edges_v[pl.ds(k * 16, 16)]
            cnts, last = plsc.scan_count(v)
            plsc.addupdate_scatter(
                hist,
                [lax.shift_right_logical(v, 7), v & 127],
                cnts.astype(jnp.float32),
                mask=last,
            )
        pltpu.sync_copy(hist, out_hbm.at[wid])

    count_hist(src_v, dego_hbm)
    count_hist(dst_v, degi_hbm)

    trash = BLK + (s % (ACC_R - BLK))
    lanes = lax.iota(jnp.int32, 16)
    cnt = jnp.int32(0)
    prev_chunks = jnp.int32(0)
    counts_vec = jnp.zeros((16,), jnp.int32)
    for p in range(NPASS):
        lo = jnp.int32(p * BLK)

        def scan_one(k, cnt):
            d = dst_v[pl.ds(k * 16, 16)]
            sv = src_v[pl.ds(k * 16, 16)]
            dl = d - lo
            m = (dl >= 0) & (dl < BLK)
            mi = m.astype(jnp.int32)
            pos = cnt + plsc.cumsum(mi) - mi
            pr = pos // CHUNK
            pc = pos - pr * CHUNK
            plsc.store_scatter(stg_s, [pr, pc], sv, mask=m)
            plsc.store_scatter(stg_d, [pr, pc], dl, mask=m)
            return cnt + jnp.sum(mi)

        def scan(k, cnt):
            for u in range(4):
                cnt = scan_one(k * 4 + u, cnt)
            return cnt

        cnt = lax.fori_loop(0, EPT // 64, scan, cnt)
        for k in range(EPT // 64 * 4, EPT // 16):
            cnt = scan_one(k, cnt)
        for t in range(CHUNK // 16):
            pos = cnt + t * 16 + lanes
            pr = pos // CHUNK
            pc = pos - pr * CHUNK
            plsc.store_scatter(stg_s, [pr, pc], s * CHUNK + t * 16 + lanes)
            plsc.store_scatter(stg_d, [pr, pc],
                               jnp.full((16,), trash, jnp.int32))
        nch = (cnt + (CHUNK - 1)) // CHUNK
        counts_vec = jnp.where(lanes == p, nch - prev_chunks, counts_vec)
        prev_chunks = nch
        cnt = nch * CHUNK
    cnt_v[0, :] = counts_vec

    pltpu.sync_copy(stg_s, psrc_hbm.at[wid])
    pltpu.sync_copy(stg_d, pdst_hbm.at[wid])
    pltpu.sync_copy(cnt_v, cnt_hbm.at[wid])


@functools.partial(
    pl.kernel,
    mesh=_mesh,
    compiler_params=_sc_params,
    out_type=jax.ShapeDtypeStruct((NC * NPASS * BLK, D), jnp.float32),
    scratch_types=[
        pltpu.VMEM((CAPC, CHUNK), jnp.int32),
        pltpu.VMEM((CAPC, CHUNK), jnp.int32),
        pltpu.VMEM((1, 16), jnp.int32),
        pltpu.VMEM((CHUNK, D), jnp.float32),
        pltpu.VMEM((CHUNK, D), jnp.float32),
        pltpu.VMEM((CHUNK, D), jnp.float32),
        pltpu.VMEM((8, D), jnp.float32),
        pltpu.VMEM_SHARED((ACC_R, D), jnp.float32),
        pltpu.SemaphoreType.DMA,
        pltpu.SemaphoreType.DMA,
        pltpu.SemaphoreType.DMA,
        pltpu.SemaphoreType.DMA,
        pltpu.SemaphoreType.DMA,
        pltpu.SemaphoreType.DMA,
    ],
)
def _agg_kernel(h_hbm, psrc_hbm, pdst_hbm, cnt_hbm, out_hbm,
                src_v, dst_v, cnt_v, rows_a, rows_b, rows_c, zbuf, acc,
                sg_a, sg_b, sg_c, ss_a, ss_b, ss_c):
    c = lax.axis_index("c")
    s = lax.axis_index("s")
    wid = c * NS + s

    def zfill(k, _):
        zbuf[k // 8, pl.ds((k % 8) * 16, 16)] = jnp.zeros((16,), jnp.float32)
        return 0

    lax.fori_loop(0, 8 * D // 16, zfill, 0)

    pltpu.sync_copy(psrc_hbm.at[wid], src_v)
    pltpu.sync_copy(pdst_hbm.at[wid], dst_v)
    pltpu.sync_copy(cnt_hbm.at[wid], cnt_v)

    slab0 = s * 128
    nslab8 = jnp.where(s < 15, 16, 15)

    def zslab(i, _):
        pltpu.sync_copy(zbuf, acc.at[pl.ds(slab0 + i * 8, 8)])
        return 0

    def zero_slab():
        lax.fori_loop(0, nslab8, zslab, 0)

    zero_slab()
    plsc.subcore_barrier()

    off = jnp.int32(0)
    cv = cnt_v[0, :]
    bufs = (rows_a, rows_b, rows_c)
    sgs = (sg_a, sg_b, sg_c)
    sss = (ss_a, ss_b, ss_c)

    def gather_start(j, q):
        pltpu.async_copy(h_hbm.at[src_v.at[j]], bufs[q], sgs[q])

    def gather_wait(j, q):
        pltpu.make_async_copy(h_hbm.at[src_v.at[j]], bufs[q], sgs[q]).wait()

    def scatter_start(j, q):
        pltpu.async_copy(bufs[q], acc.at[dst_v.at[j]], sss[q], add=True)

    def scatter_wait(j, q):
        pltpu.make_async_copy(bufs[q], acc.at[dst_v.at[j]], sss[q]).wait()

    for p in range(NPASS):
        nch = cv[p]

        @pl.when(nch > 0)
        def _():
            gather_start(off, 0)

        @pl.when(nch > 1)
        def _():
            gather_start(off + 1, 1)

        def body(jj, off):
            j = off + jj

            def step(q):
                gather_wait(j, (q + 1) % 3)

                @pl.when(jj + 2 < nch)
                def _():
                    @pl.when(jj >= 1)
                    def _():
                        scatter_wait(j - 1, q)

                    gather_start(j + 2, q)

                scatter_start(j, (q + 1) % 3)

            @pl.when(jj % 3 == 0)
            def _():
                step(2)

            @pl.when(jj % 3 == 1)
            def _():
                step(0)

            @pl.when(jj % 3 == 2)
            def _():
                step(1)

            return off

        lax.fori_loop(0, nch, body, off)

        for k in range(3):
            @pl.when(nch >= k + 1)
            def _():
                j = off + nch - 1 - k

                @pl.when((nch - 1 - k) % 3 == 0)
                def _():
                    scatter_wait(j, 0)

                @pl.when((nch - 1 - k) % 3 == 1)
                def _():
                    scatter_wait(j, 1)

                @pl.when((nch - 1 - k) % 3 == 2)
                def _():
                    scatter_wait(j, 2)

        off = off + nch
        plsc.subcore_barrier()

        @pl.when(s < 15)
        def _():
            pltpu.sync_copy(
                acc.at[pl.ds(slab0, 128)],
                out_hbm.at[pl.ds(c * NPASS * BLK + p * BLK + slab0, 128)])

        @pl.when(s == 15)
        def _():
            pltpu.sync_copy(
                acc.at[pl.ds(slab0, 120)],
                out_hbm.at[pl.ds(c * NPASS * BLK + p * BLK + slab0, 120)])

        plsc.subcore_barrier()
        if p != NPASS - 1:
            zero_slab()
            plsc.subcore_barrier()


def _elu(t):
    return jnp.where(t > 0, t, jnp.exp(jnp.minimum(t, 0.0)) - 1.0)


def _deg_reduce_body(po_ref, pi_ref, do_ref, di_ref):
    acc_o = jnp.zeros((HR, 128), jnp.float32)
    acc_i = jnp.zeros((HR, 128), jnp.float32)
    for w in range(NW):
        acc_o = acc_o + po_ref[pl.ds(w * HR, HR), :]
        acc_i = acc_i + pi_ref[pl.ds(w * HR, HR), :]
    do_ref[...] = lax.rsqrt(jnp.maximum(acc_o, 1.0))
    di_ref[...] = lax.rsqrt(jnp.maximum(acc_i, 1.0))


_deg_reduce = pl.pallas_call(
    _deg_reduce_body,
    grid=(1,),
    in_specs=[pl.BlockSpec((NW * HR, 128), lambda i: (0, 0)),
              pl.BlockSpec((NW * HR, 128), lambda i: (0, 0))],
    out_specs=[pl.BlockSpec((HR, 128), lambda i: (0, 0)),
               pl.BlockSpec((HR, 128), lambda i: (0, 0))],
    out_shape=[jax.ShapeDtypeStruct((HR, 128), jnp.float32),
               jax.ShapeDtypeStruct((HR, 128), jnp.float32)],
)


def _pre_mm_body(x_ref, do_ref, w_ref, o_ref):
    o_ref[...] = jnp.dot(x_ref[...] * do_ref[...], w_ref[...],
                         preferred_element_type=jnp.float32)


def _mid_mm_body(a0, a1, di_ref, do_ref, b_ref, w_ref, o_ref):
    agg = a0[...] + a1[...]
    t = agg * di_ref[...] + b_ref[...]
    h = _elu(t)
    o_ref[...] = jnp.dot(h * do_ref[...], w_ref[...],
                         preferred_element_type=jnp.float32)


def _final_body(a0, a1, di_ref, b_ref, o_ref):
    agg = a0[...] + a1[...]
    o_ref[...] = _elu(agg * di_ref[...] + b_ref[...])


_BLKR = 1000
_row_spec = pl.BlockSpec((_BLKR, D), lambda i: (i, 0))
_col_spec = pl.BlockSpec((_BLKR, 1), lambda i: (i, 0))
_w_spec = pl.BlockSpec((D, D), lambda i: (0, 0))
_b_spec = pl.BlockSpec((1, D), lambda i: (0, 0))
_out_sds = jax.ShapeDtypeStruct((N, D), jnp.float32)

_pre_mm = pl.pallas_call(
    _pre_mm_body,
    grid=(N // _BLKR,),
    in_specs=[_row_spec, _col_spec, _w_spec],
    out_specs=_row_spec,
    out_shape=_out_sds,
)

_mid_mm = pl.pallas_call(
    _mid_mm_body,
    grid=(N // _BLKR,),
    in_specs=[_row_spec, _row_spec, _col_spec, _col_spec, _b_spec, _w_spec],
    out_specs=_row_spec,
    out_shape=_out_sds,
)

_final = pl.pallas_call(
    _final_body,
    grid=(N // _BLKR,),
    in_specs=[_row_spec, _row_spec, _col_spec, _b_spec],
    out_specs=_row_spec,
    out_shape=_out_sds,
)


def kernel(features, edge_index, W1, b1, W2, b2):
    dego, degi, psrc, pdst, cnts = _prep_kernel(edge_index[0], edge_index[1])

    do_t, di_t = _deg_reduce(dego.reshape(NW * HR, 128),
                             degi.reshape(NW * HR, 128))
    do = do_t.reshape(NP, 1)[:N]
    di = di_t.reshape(NP, 1)[:N]

    h1 = _pre_mm(features, do, W1)
    a1 = _agg_kernel(h1, psrc, pdst, cnts)
    spc = NPASS * BLK
    h2 = _mid_mm(a1[:N], a1[spc:spc + N], di, do, b1.reshape(1, D), W2)
    a2 = _agg_kernel(h2, psrc, pdst, cnts)
    out = _final(a2[:N], a2[spc:spc + N], di, b2.reshape(1, D))
    return out
